# Initial kernel scaffold; baseline (speedup 1.0000x reference)
import jax
import jax.numpy as jnp
from jax.experimental import pallas as pl


def kernel(x, rel_k):
    raise NotImplementedError("write your pallas kernel here")


# speedup vs baseline: 1.3410x; 1.3410x over previous
import functools

import jax
import jax.numpy as jnp
from jax.experimental import pallas as pl
from jax.experimental.pallas import tpu as pltpu


def _relative_counts(S: int, M: int) -> jnp.ndarray:
    R = 2 * M + 1
    if M == 0:
        return jnp.full((S, 1), float(S), jnp.float32)
    i = jnp.arange(S)[:, None]
    d = jnp.arange(R)[None, :] - M
    j = i - d
    counts = ((j >= 0) & (j < S)).astype(jnp.float32)
    left = jnp.maximum(0, S - i[:, 0] - M).astype(jnp.float32)
    right = jnp.maximum(0, i[:, 0] - M + 1).astype(jnp.float32)
    counts = counts.at[:, 0].set(left)
    counts = counts.at[:, R - 1].set(right)
    return counts


def _pe_rel_body(x_ref, c_ref, rk_ref, o_ref, bias_ref):
    @pl.when(pl.program_id(1) == 0)
    def _():
        bias_ref[...] = jnp.dot(c_ref[...], rk_ref[...],
                                preferred_element_type=jnp.float32)

    o_ref[...] = x_ref[...] + bias_ref[...]


@functools.partial(jax.jit, static_argnames=("max_rel_dist",))
def _pe_relative(x, rel_k, *, max_rel_dist):
    B, S, D = x.shape
    M = max_rel_dist
    R = 2 * M + 1

    counts = _relative_counts(S, M)
    x2d = x.reshape(B * S, D)

    cores = 2 if B % 2 == 0 else 1
    per_core = B // cores

    out = pl.pallas_call(
        _pe_rel_body,
        out_shape=jax.ShapeDtypeStruct((B * S, D), x.dtype),
        grid_spec=pltpu.PrefetchScalarGridSpec(
            num_scalar_prefetch=0,
            grid=(cores, per_core),
            in_specs=[
                pl.BlockSpec((S, D), lambda c, b, n=per_core: (c * n + b, 0)),
                pl.BlockSpec((S, R), lambda c, b: (0, 0)),
                pl.BlockSpec((R, D), lambda c, b: (0, 0)),
            ],
            out_specs=pl.BlockSpec((S, D), lambda c, b, n=per_core: (c * n + b, 0)),
            scratch_shapes=[pltpu.VMEM((S, D), jnp.float32)],
        ),
        compiler_params=pltpu.CompilerParams(
            dimension_semantics=("parallel", "arbitrary")),
    )(x2d, counts, rel_k.astype(jnp.float32))
    return out.reshape(B, S, D)


def kernel(x, rel_k):
    return _pe_relative(x, rel_k, max_rel_dist=128)


# speedup vs baseline: 1.4660x; 1.0932x over previous
import functools

import jax
import jax.numpy as jnp
from jax.experimental import pallas as pl
from jax.experimental.pallas import tpu as pltpu


def _relative_counts(S: int, M: int) -> jnp.ndarray:
    R = 2 * M + 1
    if M == 0:
        return jnp.full((S, 1), float(S), jnp.float32)
    i = jnp.arange(S)[:, None]
    d = jnp.arange(R)[None, :] - M
    j = i - d
    counts = ((j >= 0) & (j < S)).astype(jnp.float32)
    left = jnp.maximum(0, S - i[:, 0] - M).astype(jnp.float32)
    right = jnp.maximum(0, i[:, 0] - M + 1).astype(jnp.float32)
    counts = counts.at[:, 0].set(left)
    counts = counts.at[:, R - 1].set(right)
    return counts


def _pe_rel_body(x_ref, c_ref, rk_ref, o_ref, bias_ref):
    @pl.when(pl.program_id(1) == 0)
    def _():
        bias_ref[...] = jnp.dot(c_ref[...], rk_ref[...],
                                preferred_element_type=jnp.float32)

    o_ref[...] = x_ref[...] + bias_ref[...]


@functools.partial(jax.jit, static_argnames=("max_rel_dist", "batch_per_block"))
def _pe_relative(x, rel_k, *, max_rel_dist, batch_per_block=2):
    B, S, D = x.shape
    M = max_rel_dist
    R = 2 * M + 1

    counts = _relative_counts(S, M)
    x2d = x.reshape(B * S, D)

    nb = batch_per_block
    while B % (2 * nb) != 0:
        nb -= 1
    rows = nb * S
    counts = jnp.tile(counts, (nb, 1))
    per_core = B // (2 * nb)

    out = pl.pallas_call(
        _pe_rel_body,
        out_shape=jax.ShapeDtypeStruct((B * S, D), x.dtype),
        grid_spec=pltpu.PrefetchScalarGridSpec(
            num_scalar_prefetch=0,
            grid=(2, per_core),
            in_specs=[
                pl.BlockSpec((rows, D), lambda c, b, n=per_core: (c * n + b, 0)),
                pl.BlockSpec((rows, R), lambda c, b: (0, 0)),
                pl.BlockSpec((R, D), lambda c, b: (0, 0)),
            ],
            out_specs=pl.BlockSpec((rows, D), lambda c, b, n=per_core: (c * n + b, 0)),
            scratch_shapes=[pltpu.VMEM((rows, D), jnp.float32)],
        ),
        compiler_params=pltpu.CompilerParams(
            dimension_semantics=("parallel", "arbitrary")),
    )(x2d, counts, rel_k.astype(jnp.float32))
    return out.reshape(B, S, D)


def kernel(x, rel_k):
    return _pe_relative(x, rel_k, max_rel_dist=128)
